# baseline (speedup 1.0000x reference)
import functools

import jax
import jax.numpy as jnp
from jax import lax
from jax.experimental import pallas as pl
from jax.experimental.pallas import tpu as pltpu
from jax.experimental.pallas import tpu_sc as plsc

_NI = 8
_NEG = -1e30
_LOG2E = 1.4426950408889634


def _elu(y):
    return jnp.maximum(y, 0.0) + jnp.exp2(jnp.minimum(y, 0.0) * _LOG2E) - 1.0


def _sigmoid(x):
    return 1.0 / (1.0 + jnp.exp(-x))


def _dotT(a, b):
    return lax.dot_general(a, b, (((1,), (1,)), ((), ())),
                           preferred_element_type=jnp.float32)


def _frontend_body(tokens_ref, vocab_ref, pemb_ref, tagdef_ref, tagW_ref,
                   wih_ref, whh_ref, bih_ref, bhh_ref,
                   dwih_ref, dwhh_ref, dbih_ref, dbhh_ref, npg_ref,
                   enc_ref, instr_ref, foo_ref, rs_ref, d0_ref):
    Bq, L, D = tokens_ref.shape
    T = tokens_ref[...].reshape(Bq * L, D)
    vocab = vocab_ref[...]
    A = jnp.dot(T, tagW_ref[...], preferred_element_type=jnp.float32)
    logits = _dotT(A, vocab)
    ldef = _dotT(A, tagdef_ref[...].reshape(1, D))
    m = jnp.maximum(jnp.max(logits, axis=1, keepdims=True), ldef)
    e1 = jnp.exp(logits - m)
    e2 = jnp.exp(ldef - m)
    z = jnp.sum(e1, axis=1, keepdims=True) + e2
    tagged = (e2 / z) * T + jnp.dot(e1 / z, vocab,
                                    preferred_element_type=jnp.float32)
    t3 = tagged.reshape(Bq, L, D)
    wih = wih_ref[...]
    whh = whh_ref[...]
    bias = bih_ref[...].reshape(1, 4 * D) + bhh_ref[...].reshape(1, 4 * D)
    h = jnp.zeros((Bq, D), jnp.float32)
    c = jnp.zeros((Bq, D), jnp.float32)
    for l in range(L):
        x = t3[:, l, :].reshape(Bq, D)
        g = _dotT(x, wih) + bias + _dotT(h, whh)
        gi = g[:, 0:D]
        gf = g[:, D:2 * D]
        gg = g[:, 2 * D:3 * D]
        go = g[:, 3 * D:4 * D]
        c = _sigmoid(gf) * c + _sigmoid(gi) * jnp.tanh(gg)
        h = _sigmoid(go) * jnp.tanh(c)
    enc_ref[...] = h
    dwhh = dwhh_ref[...]
    pre = _dotT(h, dwih_ref[...]) + dbih_ref[...].reshape(1, D) \
        + dbhh_ref[...].reshape(1, D)
    pemb = pemb_ref[...]
    Pp1 = pemb.shape[0]
    rowg = lax.broadcasted_iota(jnp.int32, (Bq, Bq * L), 0)
    colg = lax.broadcasted_iota(jnp.int32, (Bq, Bq * L), 1) // L
    msk = rowg == colg
    hx = jnp.zeros((Bq, D), jnp.float32)
    ones16 = jnp.ones((Bq, 16), jnp.float32)
    for t in range(_NI):
        hx = jnp.maximum(pre + _dotT(hx, dwhh), 0.0)
        lg = jnp.where(msk, _dotT(hx, tagged), _NEG)
        mm = jnp.max(lg, axis=1, keepdims=True)
        ee = jnp.exp(lg - mm)
        att = ee / jnp.sum(ee, axis=1, keepdims=True)
        instr = jnp.dot(att, tagged, preferred_element_type=jnp.float32)
        instr_ref[t] = instr
        fl = _dotT(instr, pemb)
        fm = jnp.max(fl, axis=1, keepdims=True)
        fe = jnp.exp(fl - fm)
        foo = fe / jnp.sum(fe, axis=1, keepdims=True)
        foo_ref[t] = jnp.concatenate(
            [foo, jnp.zeros((Bq, 16 - Pp1), jnp.float32)], axis=1)
        rs_ref[t] = foo[:, Pp1 - 1:Pp1] * ones16
    d0_ref[...] = (1.0 / npg_ref[...]).reshape(Bq, 1) * ones16


def _fused_body(tokens_ref, vocab_ref, pemb_ref, tagdef_ref, tagW_ref,
                wih_ref, whh_ref, bih_ref, bhh_ref,
                dwih_ref, dwhh_ref, dbih_ref, dbhh_ref, npg_ref,
                ea_ref, we_ref, na_ref, wnp_ref, wrel_ref, wnode_ref,
                v_ref, s_ref, aggv_ref, enc_ref, instr_ref, foo_ref,
                rs_ref, d0_ref, *, npgp):
    @pl.when(pl.program_id(0) == 0)
    def _():
        _frontend_body(tokens_ref, vocab_ref, pemb_ref, tagdef_ref, tagW_ref,
                       wih_ref, whh_ref, bih_ref, bhh_ref,
                       dwih_ref, dwhh_ref, dbih_ref, dbhh_ref, npg_ref,
                       enc_ref, instr_ref, foo_ref, rs_ref, d0_ref)

    _edge_part(ea_ref, we_ref, instr_ref, wrel_ref, v_ref)
    _node_part(na_ref, wnp_ref, instr_ref, foo_ref, wnode_ref, s_ref,
               aggv_ref, npgp=npgp)


def _edge_part(ea_ref, we_ref, instr_ref, wrel_ref, v_ref):
    g = pl.program_id(0)
    ea = ea_ref[...]
    EPG = ea.shape[0]
    ex2 = _dotT(ea, we_ref[...] * _LOG2E)
    wrel = wrel_ref[...].reshape(1, -1)
    sumw = jnp.sum(wrel)
    ln2 = jnp.bfloat16(1.0 / _LOG2E)
    exb = ex2.astype(jnp.bfloat16)
    rows = []
    for t in range(_NI):
        inst = instr_ref[t, pl.ds(g, 1), :].astype(jnp.bfloat16)
        y2 = exb * inst
        a = ln2 * jnp.maximum(y2, 0) + jnp.exp2(jnp.minimum(y2, 0))
        rows.append(_dotT(wrel.astype(jnp.bfloat16), a) - sumw)
    v_ref[...] = jnp.concatenate(rows, axis=0).reshape(1, _NI, EPG)


def _node_part(na_ref, wnp_ref, instr_ref, foo_ref, wnode_ref, s_ref,
               aggv_ref, *, npgp):
    g = pl.program_id(0)
    na = na_ref[...]
    NPG, P, D = na.shape
    wx = [_dotT(na[:, p, :].reshape(NPG, D), wnp_ref[p]) for p in range(P)]
    wxb = [w.astype(jnp.bfloat16) for w in wx]
    wnode = wnode_ref[...].reshape(1, D).astype(jnp.bfloat16)
    sumw = jnp.sum(wnode_ref[...])
    ln2 = jnp.bfloat16(1.0 / _LOG2E)
    l2e = jnp.bfloat16(_LOG2E)
    rows = []
    for t in range(_NI):
        fv = foo_ref[t, pl.ds(g, 1), :].astype(jnp.bfloat16)
        mt = wxb[0] * fv[0:1, 0:1]
        for p in range(1, P):
            mt = mt + wxb[p] * fv[0:1, p:p + 1]
        inst = instr_ref[t, pl.ds(g, 1), :].astype(jnp.bfloat16)
        y2 = (inst * l2e) * mt
        a = ln2 * jnp.maximum(y2, 0) + jnp.exp2(jnp.minimum(y2, 0))
        rows.append(_dotT(wnode, a) - sumw)
    S = jnp.concatenate(rows, axis=0)
    S = jnp.concatenate(
        [S, jnp.full((_NI, npgp - NPG), _NEG, jnp.float32)], axis=1)
    s_ref[...] = S.reshape(1, _NI, npgp)
    fv = foo_ref[_NI - 1, pl.ds(g, 1), :]
    aggv = na[:, 0, :].reshape(NPG, D) * fv[0:1, 0:1]
    for p in range(1, P):
        aggv = aggv + na[:, p, :].reshape(NPG, D) * fv[0:1, p:p + 1]
    aggv = jnp.concatenate(
        [aggv, jnp.zeros((npgp - NPG, D), jnp.float32)], axis=0)
    aggv_ref[...] = aggv.reshape(1, npgp, D)


def _final_body(dist_ref, aggv_ref, enc_ref, linw_ref, linb_ref, out_ref):
    dist = dist_ref[...]
    aggv = aggv_ref[...]
    agg = jnp.sum(dist[:, :, None] * aggv, axis=1)
    x = jnp.concatenate([enc_ref[...], agg], axis=1)
    out_ref[...] = _dotT(x, linw_ref[...]) + linb_ref[...].reshape(1, -1)


def _make_sc_loop(B, NPG, NPGP, EPG):
    info = plsc.get_sparse_core_info()
    NC, NS = info.num_cores, info.num_subcores
    NW = NC * NS
    NREP = -(-B // NW)
    CH = EPG // 16
    NVR = NPGP // 16
    NFULL = NPG // 16
    REM = NPG - NFULL * 16
    mesh = plsc.VectorSubcoreMesh(core_axis_name="c", subcore_axis_name="s")

    def _softmax_row(ref, t, nvr):
        m = ref[t, pl.ds(0, 16)]
        for j in range(1, nvr):
            m = jnp.maximum(m, ref[t, pl.ds(16 * j, 16)])
        mx = jnp.max(m)
        zv = jnp.zeros((16,), jnp.float32)
        for j in range(nvr):
            e = jnp.exp(ref[t, pl.ds(16 * j, 16)] - mx)
            ref[t, pl.ds(16 * j, 16)] = e
            zv = zv + e
        return 1.0 / jnp.broadcast_to(jnp.sum(zv), (16,))

    def body(v_hbm, ei_hbm, s_hbm, rs_hbm, d0_hbm, out_hbm,
             v_v, src_v, dst_v, s_v, rs_v, d0_v, dist_v, r_v, sem):
        wid = lax.axis_index("s") * NC + lax.axis_index("c")
        lane = lax.iota(jnp.int32, 16)
        zero16 = jnp.zeros((16,), jnp.float32)
        neg16 = jnp.full((16,), _NEG, jnp.float32)
        for rep in range(NREP):
            gval = wid + rep * NW

            @pl.when(gval < B)
            def _process():
                base = jnp.broadcast_to(gval * NPG, (16,)).astype(jnp.int32)
                cp_v = pltpu.async_copy(v_hbm.at[gval], v_v, sem)
                cp_s = pltpu.async_copy(ei_hbm.at[0, gval], src_v, sem)
                cp_d = pltpu.async_copy(ei_hbm.at[1, gval], dst_v, sem)
                pltpu.sync_copy(s_hbm.at[gval], s_v)
                pltpu.sync_copy(rs_hbm.at[gval], rs_v)
                pltpu.sync_copy(d0_hbm.at[gval], d0_v)
                for t in range(_NI):
                    inv = _softmax_row(s_v, t, NVR)
                    for j in range(NVR):
                        s_v[t, pl.ds(16 * j, 16)] = \
                            s_v[t, pl.ds(16 * j, 16)] * inv
                d0 = d0_v[...]
                for j in range(NVR):
                    if j < NFULL:
                        dist_v[pl.ds(16 * j, 16)] = d0
                    elif j == NFULL and REM:
                        dist_v[pl.ds(16 * j, 16)] = \
                            jnp.where(lane < REM, d0, 0.0)
                    else:
                        dist_v[pl.ds(16 * j, 16)] = zero16
                cp_v.wait()
                cp_s.wait()
                cp_d.wait()
                for t in range(_NI):
                    for j in range(NVR):
                        if j < NFULL:
                            r_v[pl.ds(16 * j, 16)] = zero16
                        elif j == NFULL and REM:
                            r_v[pl.ds(16 * j, 16)] = \
                                jnp.where(lane < REM, 0.0, _NEG)
                        else:
                            r_v[pl.ds(16 * j, 16)] = neg16

                    @plsc.parallel_loop(0, CH, unroll=8)
                    def _chunk(i):
                        si = src_v[pl.ds(i * 16, 16)] - base
                        di = dst_v[pl.ds(i * 16, 16)] - base
                        vv = v_v[t, pl.ds(i * 16, 16)]
                        gd = plsc.load_gather(dist_v, [si])
                        plsc.addupdate_scatter(r_v, [di], gd * vv)
                    m = r_v[pl.ds(0, 16)]
                    for j in range(1, NVR):
                        m = jnp.maximum(m, r_v[pl.ds(16 * j, 16)])
                    mx = jnp.max(m)
                    zv = jnp.zeros((16,), jnp.float32)
                    for j in range(NVR):
                        e = jnp.exp(r_v[pl.ds(16 * j, 16)] - mx)
                        r_v[pl.ds(16 * j, 16)] = e
                        zv = zv + e
                    inv = 1.0 / jnp.broadcast_to(jnp.sum(zv), (16,))
                    rsv = rs_v[t]
                    for j in range(NVR):
                        pr = r_v[pl.ds(16 * j, 16)] * inv
                        ps = s_v[t, pl.ds(16 * j, 16)]
                        dist_v[pl.ds(16 * j, 16)] = \
                            rsv * pr + (1.0 - rsv) * ps
                pltpu.sync_copy(dist_v, out_hbm.at[gval])

    return pl.kernel(
        body,
        out_type=jax.ShapeDtypeStruct((B, NPGP), jnp.float32),
        mesh=mesh,
        compiler_params=pltpu.CompilerParams(needs_layout_passes=False),
        scratch_types=[
            pltpu.VMEM((_NI, EPG), jnp.float32),
            pltpu.VMEM((EPG,), jnp.int32),
            pltpu.VMEM((EPG,), jnp.int32),
            pltpu.VMEM((_NI, NPGP), jnp.float32),
            pltpu.VMEM((_NI, 16), jnp.float32),
            pltpu.VMEM((16,), jnp.float32),
            pltpu.VMEM((NPGP,), jnp.float32),
            pltpu.VMEM((NPGP,), jnp.float32),
            pltpu.SemaphoreType.DMA,
        ],
    )


def kernel(node_attrs, edge_attrs, edge_indices, node_indices,
           edge_batch_indices, nodes_per_graph, tokens, concept_vocabulary,
           property_embeddings, tag_default, tag_W, lstm_Wih, lstm_Whh,
           lstm_bih, lstm_bhh, dec_Wih, dec_Whh, dec_bih, dec_bhh,
           W_np, W_edge, w_node_score, w_rel_score, lin_W, lin_b):
    B, L, D = tokens.shape
    N, P, _ = node_attrs.shape
    E = edge_attrs.shape[0]
    V = concept_vocabulary.shape[0]
    OUT = lin_W.shape[0]
    NPG = N // B
    EPG = E // B
    NPGP = ((NPG + 127) // 128) * 128

    full = lambda *shape: pl.BlockSpec(shape, lambda g: tuple(0 for _ in shape))
    v, s, aggv, enc, instr, foo, rs, d0 = pl.pallas_call(
        functools.partial(_fused_body, npgp=NPGP),
        grid=(B,),
        in_specs=[
            full(B, L, D), full(V, D), full(P + 1, D), full(D), full(D, D),
            full(4 * D, D), full(4 * D, D), full(4 * D), full(4 * D),
            full(D, D), full(D, D), full(D), full(D), full(B),
            pl.BlockSpec((EPG, D), lambda g: (g, 0)),
            full(D, D),
            pl.BlockSpec((NPG, P, D), lambda g: (g, 0, 0)),
            full(P, D, D),
            full(D),
            full(D),
        ],
        out_specs=[
            pl.BlockSpec((1, _NI, EPG), lambda g: (g, 0, 0)),
            pl.BlockSpec((1, _NI, NPGP), lambda g: (g, 0, 0)),
            pl.BlockSpec((1, NPGP, D), lambda g: (g, 0, 0)),
            full(B, D), full(_NI, B, D), full(_NI, B, 16),
            full(_NI, B, 16), full(B, 16),
        ],
        out_shape=[
            jax.ShapeDtypeStruct((B, _NI, EPG), jnp.float32),
            jax.ShapeDtypeStruct((B, _NI, NPGP), jnp.float32),
            jax.ShapeDtypeStruct((B, NPGP, D), jnp.float32),
            jax.ShapeDtypeStruct((B, D), jnp.float32),
            jax.ShapeDtypeStruct((_NI, B, D), jnp.float32),
            jax.ShapeDtypeStruct((_NI, B, 16), jnp.float32),
            jax.ShapeDtypeStruct((_NI, B, 16), jnp.float32),
            jax.ShapeDtypeStruct((B, 16), jnp.float32),
        ],
    )(tokens, concept_vocabulary, property_embeddings, tag_default, tag_W,
      lstm_Wih, lstm_Whh, lstm_bih, lstm_bhh,
      dec_Wih, dec_Whh, dec_bih, dec_bhh, nodes_per_graph,
      edge_attrs, W_edge, node_attrs, W_np, w_rel_score, w_node_score)

    ei3 = edge_indices.reshape(2, B, EPG)
    rs_sc = jnp.transpose(rs, (1, 0, 2))

    dist = _make_sc_loop(B, NPG, NPGP, EPG)(v, ei3, s, rs_sc, d0)

    out = pl.pallas_call(
        _final_body,
        out_shape=jax.ShapeDtypeStruct((B, OUT), jnp.float32),
    )(dist, aggv, enc, lin_W, lin_b)
    return out


# speedup vs baseline: 1.0079x; 1.0079x over previous
import functools

import jax
import jax.numpy as jnp
from jax import lax
from jax.experimental import pallas as pl
from jax.experimental.pallas import tpu as pltpu
from jax.experimental.pallas import tpu_sc as plsc

_NI = 8
_NEG = -1e30
_LOG2E = 1.4426950408889634


def _elu(y):
    return jnp.maximum(y, 0.0) + jnp.exp2(jnp.minimum(y, 0.0) * _LOG2E) - 1.0


def _sigmoid(x):
    return 1.0 / (1.0 + jnp.exp(-x))


def _dotT(a, b):
    return lax.dot_general(a, b, (((1,), (1,)), ((), ())),
                           preferred_element_type=jnp.float32)


def _frontend_body(tokens_ref, vocab_ref, pemb_ref, tagdef_ref, tagW_ref,
                   wih_ref, whh_ref, bih_ref, bhh_ref,
                   dwih_ref, dwhh_ref, dbih_ref, dbhh_ref, npg_ref,
                   enc_ref, instr_ref, foo_ref, rs_ref, d0_ref):
    Bq, L, D = tokens_ref.shape
    T = tokens_ref[...].reshape(Bq * L, D)
    vocab = vocab_ref[...]
    A = jnp.dot(T, tagW_ref[...], preferred_element_type=jnp.float32)
    logits = _dotT(A, vocab)
    ldef = _dotT(A, tagdef_ref[...].reshape(1, D))
    m = jnp.maximum(jnp.max(logits, axis=1, keepdims=True), ldef)
    e1 = jnp.exp(logits - m)
    e2 = jnp.exp(ldef - m)
    z = jnp.sum(e1, axis=1, keepdims=True) + e2
    tagged = (e2 / z) * T + jnp.dot(e1 / z, vocab,
                                    preferred_element_type=jnp.float32)
    t3 = tagged.reshape(Bq, L, D)
    wih = wih_ref[...]
    whh = whh_ref[...]
    bias = bih_ref[...].reshape(1, 4 * D) + bhh_ref[...].reshape(1, 4 * D)
    h = jnp.zeros((Bq, D), jnp.float32)
    c = jnp.zeros((Bq, D), jnp.float32)
    for l in range(L):
        x = t3[:, l, :].reshape(Bq, D)
        g = _dotT(x, wih) + bias + _dotT(h, whh)
        gi = g[:, 0:D]
        gf = g[:, D:2 * D]
        gg = g[:, 2 * D:3 * D]
        go = g[:, 3 * D:4 * D]
        c = _sigmoid(gf) * c + _sigmoid(gi) * jnp.tanh(gg)
        h = _sigmoid(go) * jnp.tanh(c)
    enc_ref[...] = h
    dwhh = dwhh_ref[...]
    pre = _dotT(h, dwih_ref[...]) + dbih_ref[...].reshape(1, D) \
        + dbhh_ref[...].reshape(1, D)
    pemb = pemb_ref[...]
    Pp1 = pemb.shape[0]
    rowg = lax.broadcasted_iota(jnp.int32, (Bq, Bq * L), 0)
    colg = lax.broadcasted_iota(jnp.int32, (Bq, Bq * L), 1) // L
    msk = rowg == colg
    hx = jnp.zeros((Bq, D), jnp.float32)
    ones16 = jnp.ones((Bq, 16), jnp.float32)
    for t in range(_NI):
        hx = jnp.maximum(pre + _dotT(hx, dwhh), 0.0)
        lg = jnp.where(msk, _dotT(hx, tagged), _NEG)
        mm = jnp.max(lg, axis=1, keepdims=True)
        ee = jnp.exp(lg - mm)
        att = ee / jnp.sum(ee, axis=1, keepdims=True)
        instr = jnp.dot(att, tagged, preferred_element_type=jnp.float32)
        instr_ref[t] = instr
        fl = _dotT(instr, pemb)
        fm = jnp.max(fl, axis=1, keepdims=True)
        fe = jnp.exp(fl - fm)
        foo = fe / jnp.sum(fe, axis=1, keepdims=True)
        foo_ref[t] = jnp.concatenate(
            [foo, jnp.zeros((Bq, 16 - Pp1), jnp.float32)], axis=1)
        rs_ref[t] = foo[:, Pp1 - 1:Pp1] * ones16
    d0_ref[...] = (1.0 / npg_ref[...]).reshape(Bq, 1) * ones16


def _edge_node_body(ea_ref, we_ref, na_ref, wnp_ref, instr_ref, foo_ref,
                    wrel_ref, wnode_ref, v_ref, s_ref, aggv_ref, *, npgp):
    _edge_part(ea_ref, we_ref, instr_ref, wrel_ref, v_ref)
    _node_part(na_ref, wnp_ref, instr_ref, foo_ref, wnode_ref, s_ref,
               aggv_ref, npgp=npgp)


def _edge_part(ea_ref, we_ref, instr_ref, wrel_ref, v_ref):
    g = pl.program_id(0)
    ea = ea_ref[...]
    EPG = ea.shape[0]
    ex2 = _dotT(ea, we_ref[...] * _LOG2E)
    wrel = wrel_ref[...].reshape(1, -1)
    sumw = jnp.sum(wrel)
    ln2 = jnp.bfloat16(1.0 / _LOG2E)
    exb = ex2.astype(jnp.bfloat16)
    rows = []
    for t in range(_NI):
        inst = instr_ref[t, pl.ds(g, 1), :].astype(jnp.bfloat16)
        y2 = exb * inst
        a = ln2 * jnp.maximum(y2, 0) + jnp.exp2(jnp.minimum(y2, 0))
        rows.append(_dotT(wrel.astype(jnp.bfloat16), a) - sumw)
    v_ref[...] = jnp.concatenate(rows, axis=0).reshape(1, _NI, EPG)


def _node_part(na_ref, wnp_ref, instr_ref, foo_ref, wnode_ref, s_ref,
               aggv_ref, *, npgp):
    g = pl.program_id(0)
    na = na_ref[...]
    NPG, P, D = na.shape
    wx = [_dotT(na[:, p, :].reshape(NPG, D), wnp_ref[p]) for p in range(P)]
    wxb = [w.astype(jnp.bfloat16) for w in wx]
    wnode = wnode_ref[...].reshape(1, D).astype(jnp.bfloat16)
    sumw = jnp.sum(wnode_ref[...])
    ln2 = jnp.bfloat16(1.0 / _LOG2E)
    l2e = jnp.bfloat16(_LOG2E)
    rows = []
    for t in range(_NI):
        fv = foo_ref[t, pl.ds(g, 1), :].astype(jnp.bfloat16)
        mt = wxb[0] * fv[0:1, 0:1]
        for p in range(1, P):
            mt = mt + wxb[p] * fv[0:1, p:p + 1]
        inst = instr_ref[t, pl.ds(g, 1), :].astype(jnp.bfloat16)
        y2 = (inst * l2e) * mt
        a = ln2 * jnp.maximum(y2, 0) + jnp.exp2(jnp.minimum(y2, 0))
        rows.append(_dotT(wnode, a) - sumw)
    S = jnp.concatenate(rows, axis=0)
    S = jnp.concatenate(
        [S, jnp.full((_NI, npgp - NPG), _NEG, jnp.float32)], axis=1)
    s_ref[...] = S.reshape(1, _NI, npgp)
    fv = foo_ref[_NI - 1, pl.ds(g, 1), :]
    aggv = na[:, 0, :].reshape(NPG, D) * fv[0:1, 0:1]
    for p in range(1, P):
        aggv = aggv + na[:, p, :].reshape(NPG, D) * fv[0:1, p:p + 1]
    aggv = jnp.concatenate(
        [aggv, jnp.zeros((npgp - NPG, D), jnp.float32)], axis=0)
    aggv_ref[...] = aggv.reshape(1, npgp, D)


def _final_body(dist_ref, aggv_ref, enc_ref, linw_ref, linb_ref, out_ref):
    dist = dist_ref[...]
    aggv = aggv_ref[...]
    agg = jnp.sum(dist[:, :, None] * aggv, axis=1)
    x = jnp.concatenate([enc_ref[...], agg], axis=1)
    out_ref[...] = _dotT(x, linw_ref[...]) + linb_ref[...].reshape(1, -1)


def _make_sc_loop(B, NPG, NPGP, EPG):
    info = plsc.get_sparse_core_info()
    NC, NS = info.num_cores, info.num_subcores
    NW = NC * NS
    NREP = -(-B // NW)
    CH = EPG // 16
    NVR = NPGP // 16
    NFULL = NPG // 16
    REM = NPG - NFULL * 16
    mesh = plsc.VectorSubcoreMesh(core_axis_name="c", subcore_axis_name="s")

    def _softmax_row(ref, t, nvr):
        m = ref[t, pl.ds(0, 16)]
        for j in range(1, nvr):
            m = jnp.maximum(m, ref[t, pl.ds(16 * j, 16)])
        mx = jnp.max(m)
        zv = jnp.zeros((16,), jnp.float32)
        for j in range(nvr):
            e = jnp.exp(ref[t, pl.ds(16 * j, 16)] - mx)
            ref[t, pl.ds(16 * j, 16)] = e
            zv = zv + e
        return 1.0 / jnp.broadcast_to(jnp.sum(zv), (16,))

    def body(v_hbm, ei_hbm, s_hbm, rs_hbm, d0_hbm, out_hbm,
             v_v, src_v, dst_v, s_v, rs_v, d0_v, dist_v, r_v, sem):
        wid = lax.axis_index("s") * NC + lax.axis_index("c")
        lane = lax.iota(jnp.int32, 16)
        zero16 = jnp.zeros((16,), jnp.float32)
        neg16 = jnp.full((16,), _NEG, jnp.float32)
        for rep in range(NREP):
            gval = wid + rep * NW

            @pl.when(gval < B)
            def _process():
                base = jnp.broadcast_to(gval * NPG, (16,)).astype(jnp.int32)
                cp_v = pltpu.async_copy(v_hbm.at[gval], v_v, sem)
                cp_s = pltpu.async_copy(ei_hbm.at[0, gval], src_v, sem)
                cp_d = pltpu.async_copy(ei_hbm.at[1, gval], dst_v, sem)
                pltpu.sync_copy(s_hbm.at[gval], s_v)
                pltpu.sync_copy(rs_hbm.at[gval], rs_v)
                pltpu.sync_copy(d0_hbm.at[gval], d0_v)
                for t in range(_NI):
                    inv = _softmax_row(s_v, t, NVR)
                    for j in range(NVR):
                        s_v[t, pl.ds(16 * j, 16)] = \
                            s_v[t, pl.ds(16 * j, 16)] * inv
                d0 = d0_v[...]
                for j in range(NVR):
                    if j < NFULL:
                        dist_v[pl.ds(16 * j, 16)] = d0
                    elif j == NFULL and REM:
                        dist_v[pl.ds(16 * j, 16)] = \
                            jnp.where(lane < REM, d0, 0.0)
                    else:
                        dist_v[pl.ds(16 * j, 16)] = zero16
                cp_v.wait()
                cp_s.wait()
                cp_d.wait()
                for t in range(_NI):
                    for j in range(NVR):
                        if j < NFULL:
                            r_v[pl.ds(16 * j, 16)] = zero16
                        elif j == NFULL and REM:
                            r_v[pl.ds(16 * j, 16)] = \
                                jnp.where(lane < REM, 0.0, _NEG)
                        else:
                            r_v[pl.ds(16 * j, 16)] = neg16

                    @plsc.parallel_loop(0, CH, unroll=8)
                    def _chunk(i):
                        si = src_v[pl.ds(i * 16, 16)] - base
                        di = dst_v[pl.ds(i * 16, 16)] - base
                        vv = v_v[t, pl.ds(i * 16, 16)]
                        gd = plsc.load_gather(dist_v, [si])
                        plsc.addupdate_scatter(r_v, [di], gd * vv)
                    m = r_v[pl.ds(0, 16)]
                    for j in range(1, NVR):
                        m = jnp.maximum(m, r_v[pl.ds(16 * j, 16)])
                    mx = jnp.max(m)
                    zv = jnp.zeros((16,), jnp.float32)
                    for j in range(NVR):
                        e = jnp.exp(r_v[pl.ds(16 * j, 16)] - mx)
                        r_v[pl.ds(16 * j, 16)] = e
                        zv = zv + e
                    inv = 1.0 / jnp.broadcast_to(jnp.sum(zv), (16,))
                    rsv = rs_v[t]
                    for j in range(NVR):
                        pr = r_v[pl.ds(16 * j, 16)] * inv
                        ps = s_v[t, pl.ds(16 * j, 16)]
                        dist_v[pl.ds(16 * j, 16)] = \
                            rsv * pr + (1.0 - rsv) * ps
                pltpu.sync_copy(dist_v, out_hbm.at[gval])

    return pl.kernel(
        body,
        out_type=jax.ShapeDtypeStruct((B, NPGP), jnp.float32),
        mesh=mesh,
        compiler_params=pltpu.CompilerParams(needs_layout_passes=False),
        scratch_types=[
            pltpu.VMEM((_NI, EPG), jnp.float32),
            pltpu.VMEM((EPG,), jnp.int32),
            pltpu.VMEM((EPG,), jnp.int32),
            pltpu.VMEM((_NI, NPGP), jnp.float32),
            pltpu.VMEM((_NI, 16), jnp.float32),
            pltpu.VMEM((16,), jnp.float32),
            pltpu.VMEM((NPGP,), jnp.float32),
            pltpu.VMEM((NPGP,), jnp.float32),
            pltpu.SemaphoreType.DMA,
        ],
    )


def kernel(node_attrs, edge_attrs, edge_indices, node_indices,
           edge_batch_indices, nodes_per_graph, tokens, concept_vocabulary,
           property_embeddings, tag_default, tag_W, lstm_Wih, lstm_Whh,
           lstm_bih, lstm_bhh, dec_Wih, dec_Whh, dec_bih, dec_bhh,
           W_np, W_edge, w_node_score, w_rel_score, lin_W, lin_b):
    B, L, D = tokens.shape
    N, P, _ = node_attrs.shape
    E = edge_attrs.shape[0]
    V = concept_vocabulary.shape[0]
    OUT = lin_W.shape[0]
    NPG = N // B
    EPG = E // B
    NPGP = ((NPG + 127) // 128) * 128

    enc, instr, foo, rs, d0 = pl.pallas_call(
        _frontend_body,
        out_shape=[
            jax.ShapeDtypeStruct((B, D), jnp.float32),
            jax.ShapeDtypeStruct((_NI, B, D), jnp.float32),
            jax.ShapeDtypeStruct((_NI, B, 16), jnp.float32),
            jax.ShapeDtypeStruct((_NI, B, 16), jnp.float32),
            jax.ShapeDtypeStruct((B, 16), jnp.float32),
        ],
    )(tokens, concept_vocabulary, property_embeddings, tag_default, tag_W,
      lstm_Wih, lstm_Whh, lstm_bih, lstm_bhh,
      dec_Wih, dec_Whh, dec_bih, dec_bhh, nodes_per_graph)

    v, s, aggv = pl.pallas_call(
        functools.partial(_edge_node_body, npgp=NPGP),
        grid=(B,),
        in_specs=[
            pl.BlockSpec((EPG, D), lambda g: (g, 0)),
            pl.BlockSpec((D, D), lambda g: (0, 0)),
            pl.BlockSpec((NPG, P, D), lambda g: (g, 0, 0)),
            pl.BlockSpec((P, D, D), lambda g: (0, 0, 0)),
            pl.BlockSpec((_NI, B, D), lambda g: (0, 0, 0)),
            pl.BlockSpec((_NI, B, 16), lambda g: (0, 0, 0)),
            pl.BlockSpec((D,), lambda g: (0,)),
            pl.BlockSpec((D,), lambda g: (0,)),
        ],
        out_specs=[
            pl.BlockSpec((1, _NI, EPG), lambda g: (g, 0, 0)),
            pl.BlockSpec((1, _NI, NPGP), lambda g: (g, 0, 0)),
            pl.BlockSpec((1, NPGP, D), lambda g: (g, 0, 0)),
        ],
        out_shape=[
            jax.ShapeDtypeStruct((B, _NI, EPG), jnp.float32),
            jax.ShapeDtypeStruct((B, _NI, NPGP), jnp.float32),
            jax.ShapeDtypeStruct((B, NPGP, D), jnp.float32),
        ],
    )(edge_attrs, W_edge, node_attrs, W_np, instr, foo,
      w_rel_score, w_node_score)

    ei3 = edge_indices.reshape(2, B, EPG)
    rs_sc = jnp.transpose(rs, (1, 0, 2))

    dist = _make_sc_loop(B, NPG, NPGP, EPG)(v, ei3, s, rs_sc, d0)

    out = pl.pallas_call(
        _final_body,
        out_shape=jax.ShapeDtypeStruct((B, OUT), jnp.float32),
    )(dist, aggv, enc, lin_W, lin_b)
    return out


# speedup vs baseline: 1.0090x; 1.0011x over previous
import functools

import jax
import jax.numpy as jnp
from jax import lax
from jax.experimental import pallas as pl
from jax.experimental.pallas import tpu as pltpu
from jax.experimental.pallas import tpu_sc as plsc

_NI = 8
_NEG = -1e30
_LOG2E = 1.4426950408889634


def _sigmoid(x):
    return 1.0 / (1.0 + jnp.exp(-x))


def _dotT(a, b):
    return lax.dot_general(a, b, (((1,), (1,)), ((), ())),
                           preferred_element_type=jnp.float32)


def _frontend_body(tokens_ref, vocab_ref, pemb_ref, tagdef_ref, tagW_ref,
                   wih_ref, whh_ref, bih_ref, bhh_ref,
                   dwih_ref, dwhh_ref, dbih_ref, dbhh_ref, npg_ref,
                   enc_ref, instr_ref, foo_ref, rs_ref, d0_ref):
    Bq, L, D = tokens_ref.shape
    T = tokens_ref[...].reshape(Bq * L, D)
    vocab = vocab_ref[...]
    A = jnp.dot(T, tagW_ref[...], preferred_element_type=jnp.float32)
    logits = _dotT(A, vocab)
    ldef = _dotT(A, tagdef_ref[...].reshape(1, D))
    m = jnp.maximum(jnp.max(logits, axis=1, keepdims=True), ldef)
    e1 = jnp.exp(logits - m)
    e2 = jnp.exp(ldef - m)
    z = jnp.sum(e1, axis=1, keepdims=True) + e2
    tagged = (e2 / z) * T + jnp.dot(e1 / z, vocab,
                                    preferred_element_type=jnp.float32)
    t3 = tagged.reshape(Bq, L, D)
    wih = wih_ref[...]
    whh = whh_ref[...]
    bias = bih_ref[...].reshape(1, 4 * D) + bhh_ref[...].reshape(1, 4 * D)
    h = jnp.zeros((Bq, D), jnp.float32)
    c = jnp.zeros((Bq, D), jnp.float32)
    for l in range(L):
        x = t3[:, l, :].reshape(Bq, D)
        g = _dotT(x, wih) + bias + _dotT(h, whh)
        gi = g[:, 0:D]
        gf = g[:, D:2 * D]
        gg = g[:, 2 * D:3 * D]
        go = g[:, 3 * D:4 * D]
        c = _sigmoid(gf) * c + _sigmoid(gi) * jnp.tanh(gg)
        h = _sigmoid(go) * jnp.tanh(c)
    enc_ref[...] = h
    dwhh = dwhh_ref[...]
    pre = _dotT(h, dwih_ref[...]) + dbih_ref[...].reshape(1, D) \
        + dbhh_ref[...].reshape(1, D)
    pemb = pemb_ref[...]
    Pp1 = pemb.shape[0]
    rowg = lax.broadcasted_iota(jnp.int32, (Bq, Bq * L), 0)
    colg = lax.broadcasted_iota(jnp.int32, (Bq, Bq * L), 1) // L
    msk = rowg == colg
    hx = jnp.zeros((Bq, D), jnp.float32)
    ones16 = jnp.ones((Bq, 16), jnp.float32)
    for t in range(_NI):
        hx = jnp.maximum(pre + _dotT(hx, dwhh), 0.0)
        lg = jnp.where(msk, _dotT(hx, tagged), _NEG)
        mm = jnp.max(lg, axis=1, keepdims=True)
        ee = jnp.exp(lg - mm)
        att = ee / jnp.sum(ee, axis=1, keepdims=True)
        instr = jnp.dot(att, tagged, preferred_element_type=jnp.float32)
        instr_ref[t] = instr
        fl = _dotT(instr, pemb)
        fm = jnp.max(fl, axis=1, keepdims=True)
        fe = jnp.exp(fl - fm)
        foo = fe / jnp.sum(fe, axis=1, keepdims=True)
        foo_ref[t] = jnp.concatenate(
            [foo, jnp.zeros((Bq, 16 - Pp1), jnp.float32)], axis=1)
        rs_ref[t] = foo[:, Pp1 - 1:Pp1] * ones16
    d0_ref[...] = (1.0 / npg_ref[...]).reshape(Bq, 1) * ones16


def _edge_node_body(ea_ref, we_ref, na_ref, wnp_ref, instr_ref, foo_ref,
                    wrel_ref, wnode_ref, v_ref, s_ref, aggv_ref, *, npgp):
    _edge_part(ea_ref, we_ref, instr_ref, wrel_ref, v_ref)
    _node_part(na_ref, wnp_ref, instr_ref, foo_ref, wnode_ref, s_ref,
               aggv_ref, npgp=npgp)


def _edge_part(ea_ref, we_ref, instr_ref, wrel_ref, v_ref):
    g = pl.program_id(0)
    ea = ea_ref[...]
    EPG = ea.shape[0]
    ex2 = _dotT(ea, we_ref[...] * _LOG2E)
    wrel = wrel_ref[...].reshape(1, -1)
    sumw = jnp.sum(wrel)
    ln2 = jnp.bfloat16(1.0 / _LOG2E)
    exb = ex2.astype(jnp.bfloat16)
    rows = []
    for t in range(_NI):
        inst = instr_ref[t, pl.ds(g, 1), :].astype(jnp.bfloat16)
        y2 = exb * inst
        a = ln2 * jnp.maximum(y2, 0) + jnp.exp2(jnp.minimum(y2, 0))
        rows.append(_dotT(wrel.astype(jnp.bfloat16), a) - sumw)
    v_ref[...] = jnp.concatenate(rows, axis=0).reshape(1, _NI, EPG)


def _node_part(na_ref, wnp_ref, instr_ref, foo_ref, wnode_ref, s_ref,
               aggv_ref, *, npgp):
    g = pl.program_id(0)
    na = na_ref[...]
    NPG, P, D = na.shape
    wx = [_dotT(na[:, p, :].reshape(NPG, D), wnp_ref[p]) for p in range(P)]
    wxb = [w.astype(jnp.bfloat16) for w in wx]
    wnode = wnode_ref[...].reshape(1, D).astype(jnp.bfloat16)
    sumw = jnp.sum(wnode_ref[...])
    ln2 = jnp.bfloat16(1.0 / _LOG2E)
    l2e = jnp.bfloat16(_LOG2E)
    rows = []
    for t in range(_NI):
        fv = foo_ref[t, pl.ds(g, 1), :].astype(jnp.bfloat16)
        mt = wxb[0] * fv[0:1, 0:1]
        for p in range(1, P):
            mt = mt + wxb[p] * fv[0:1, p:p + 1]
        inst = instr_ref[t, pl.ds(g, 1), :].astype(jnp.bfloat16)
        y2 = (inst * l2e) * mt
        a = ln2 * jnp.maximum(y2, 0) + jnp.exp2(jnp.minimum(y2, 0))
        rows.append(_dotT(wnode, a) - sumw)
    S = jnp.concatenate(rows, axis=0)
    S = jnp.concatenate(
        [S, jnp.full((_NI, npgp - NPG), _NEG, jnp.float32)], axis=1)
    s_ref[...] = S.reshape(1, _NI, npgp)
    fv = foo_ref[_NI - 1, pl.ds(g, 1), :]
    aggv = na[:, 0, :].reshape(NPG, D) * fv[0:1, 0:1]
    for p in range(1, P):
        aggv = aggv + na[:, p, :].reshape(NPG, D) * fv[0:1, p:p + 1]
    aggv = jnp.concatenate(
        [aggv, jnp.zeros((npgp - NPG, D), jnp.float32)], axis=0)
    aggv_ref[...] = aggv.reshape(1, npgp, D)


def _final_body(dist_ref, aggv_ref, enc_ref, linw_ref, linb_ref, out_ref):
    dist = dist_ref[...]
    aggv = aggv_ref[...]
    agg = jnp.sum(dist[:, :, None] * aggv, axis=1)
    x = jnp.concatenate([enc_ref[...], agg], axis=1)
    out_ref[...] = _dotT(x, linw_ref[...]) + linb_ref[...].reshape(1, -1)


def _make_sc_loop(B, NPG, NPGP, EPG):
    info = plsc.get_sparse_core_info()
    NC, NS = info.num_cores, info.num_subcores
    NW = NC * NS
    NREP = -(-B // NW)
    CH = EPG // 16
    NVR = NPGP // 16
    NFULL = NPG // 16
    REM = NPG - NFULL * 16
    mesh = plsc.VectorSubcoreMesh(core_axis_name="c", subcore_axis_name="s")

    def _softmax_row(ref, t, nvr):
        m = ref[t, pl.ds(0, 16)]
        for j in range(1, nvr):
            m = jnp.maximum(m, ref[t, pl.ds(16 * j, 16)])
        mx = jnp.max(m)
        zv = jnp.zeros((16,), jnp.float32)
        for j in range(nvr):
            e = jnp.exp(ref[t, pl.ds(16 * j, 16)] - mx)
            ref[t, pl.ds(16 * j, 16)] = e
            zv = zv + e
        return 1.0 / jnp.broadcast_to(jnp.sum(zv), (16,))

    def body(v_hbm, ei_hbm, s_hbm, rs_hbm, d0_hbm, out_hbm,
             v_v, src_v, dst_v, s_v, rs_v, d0_v, dist_v, r_v, sem):
        wid = lax.axis_index("s") * NC + lax.axis_index("c")
        lane = lax.iota(jnp.int32, 16)
        zero16 = jnp.zeros((16,), jnp.float32)
        neg16 = jnp.full((16,), _NEG, jnp.float32)
        for rep in range(NREP):
            gval = wid + rep * NW

            @pl.when(gval < B)
            def _process():
                base = jnp.broadcast_to(gval * NPG, (16,)).astype(jnp.int32)
                cp_v = pltpu.async_copy(v_hbm.at[gval], v_v, sem)
                cp_s = pltpu.async_copy(ei_hbm.at[0, gval], src_v, sem)
                cp_d = pltpu.async_copy(ei_hbm.at[1, gval], dst_v, sem)
                pltpu.sync_copy(s_hbm.at[gval], s_v)
                pltpu.sync_copy(rs_hbm.at[gval], rs_v)
                pltpu.sync_copy(d0_hbm.at[gval], d0_v)
                for t in range(_NI):
                    inv = _softmax_row(s_v, t, NVR)
                    for j in range(NVR):
                        s_v[t, pl.ds(16 * j, 16)] = \
                            s_v[t, pl.ds(16 * j, 16)] * inv
                d0 = d0_v[...]
                for j in range(NVR):
                    if j < NFULL:
                        dist_v[pl.ds(16 * j, 16)] = d0
                    elif j == NFULL and REM:
                        dist_v[pl.ds(16 * j, 16)] = \
                            jnp.where(lane < REM, d0, 0.0)
                    else:
                        dist_v[pl.ds(16 * j, 16)] = zero16
                cp_v.wait()
                cp_s.wait()
                cp_d.wait()
                for t in range(_NI):
                    for j in range(NVR):
                        if j < NFULL:
                            r_v[pl.ds(16 * j, 16)] = zero16
                        elif j == NFULL and REM:
                            r_v[pl.ds(16 * j, 16)] = \
                                jnp.where(lane < REM, 0.0, _NEG)
                        else:
                            r_v[pl.ds(16 * j, 16)] = neg16

                    @plsc.parallel_loop(0, CH, unroll=8)
                    def _chunk(i):
                        si = src_v[pl.ds(i * 16, 16)] - base
                        di = dst_v[pl.ds(i * 16, 16)] - base
                        vv = v_v[t, pl.ds(i * 16, 16)]
                        gd = plsc.load_gather(dist_v, [si])
                        plsc.addupdate_scatter(r_v, [di], gd * vv)
                    m = r_v[pl.ds(0, 16)]
                    for j in range(1, NVR):
                        m = jnp.maximum(m, r_v[pl.ds(16 * j, 16)])
                    mx = jnp.max(m)
                    zv = jnp.zeros((16,), jnp.float32)
                    for j in range(NVR):
                        e = jnp.exp(r_v[pl.ds(16 * j, 16)] - mx)
                        r_v[pl.ds(16 * j, 16)] = e
                        zv = zv + e
                    inv = 1.0 / jnp.broadcast_to(jnp.sum(zv), (16,))
                    rsv = rs_v[t]
                    for j in range(NVR):
                        pr = r_v[pl.ds(16 * j, 16)] * inv
                        ps = s_v[t, pl.ds(16 * j, 16)]
                        dist_v[pl.ds(16 * j, 16)] = \
                            rsv * pr + (1.0 - rsv) * ps
                pltpu.sync_copy(dist_v, out_hbm.at[gval])

    return pl.kernel(
        body,
        out_type=jax.ShapeDtypeStruct((B, NPGP), jnp.float32),
        mesh=mesh,
        compiler_params=pltpu.CompilerParams(needs_layout_passes=False),
        scratch_types=[
            pltpu.VMEM((_NI, EPG), jnp.float32),
            pltpu.VMEM((EPG,), jnp.int32),
            pltpu.VMEM((EPG,), jnp.int32),
            pltpu.VMEM((_NI, NPGP), jnp.float32),
            pltpu.VMEM((_NI, 16), jnp.float32),
            pltpu.VMEM((16,), jnp.float32),
            pltpu.VMEM((NPGP,), jnp.float32),
            pltpu.VMEM((NPGP,), jnp.float32),
            pltpu.SemaphoreType.DMA,
        ],
    )


def kernel(node_attrs, edge_attrs, edge_indices, node_indices,
           edge_batch_indices, nodes_per_graph, tokens, concept_vocabulary,
           property_embeddings, tag_default, tag_W, lstm_Wih, lstm_Whh,
           lstm_bih, lstm_bhh, dec_Wih, dec_Whh, dec_bih, dec_bhh,
           W_np, W_edge, w_node_score, w_rel_score, lin_W, lin_b):
    B, L, D = tokens.shape
    N, P, _ = node_attrs.shape
    E = edge_attrs.shape[0]
    V = concept_vocabulary.shape[0]
    OUT = lin_W.shape[0]
    NPG = N // B
    EPG = E // B
    NPGP = ((NPG + 127) // 128) * 128

    enc, instr, foo, rs, d0 = pl.pallas_call(
        _frontend_body,
        out_shape=[
            jax.ShapeDtypeStruct((B, D), jnp.float32),
            jax.ShapeDtypeStruct((_NI, B, D), jnp.float32),
            jax.ShapeDtypeStruct((_NI, B, 16), jnp.float32),
            jax.ShapeDtypeStruct((_NI, B, 16), jnp.float32),
            jax.ShapeDtypeStruct((B, 16), jnp.float32),
        ],
    )(tokens, concept_vocabulary, property_embeddings, tag_default, tag_W,
      lstm_Wih, lstm_Whh, lstm_bih, lstm_bhh,
      dec_Wih, dec_Whh, dec_bih, dec_bhh, nodes_per_graph)

    v, s, aggv = pl.pallas_call(
        functools.partial(_edge_node_body, npgp=NPGP),
        grid=(B,),
        in_specs=[
            pl.BlockSpec((EPG, D), lambda g: (g, 0)),
            pl.BlockSpec((D, D), lambda g: (0, 0)),
            pl.BlockSpec((NPG, P, D), lambda g: (g, 0, 0)),
            pl.BlockSpec((P, D, D), lambda g: (0, 0, 0)),
            pl.BlockSpec((_NI, B, D), lambda g: (0, 0, 0)),
            pl.BlockSpec((_NI, B, 16), lambda g: (0, 0, 0)),
            pl.BlockSpec((D,), lambda g: (0,)),
            pl.BlockSpec((D,), lambda g: (0,)),
        ],
        out_specs=[
            pl.BlockSpec((1, _NI, EPG), lambda g: (g, 0, 0)),
            pl.BlockSpec((1, _NI, NPGP), lambda g: (g, 0, 0)),
            pl.BlockSpec((1, NPGP, D), lambda g: (g, 0, 0)),
        ],
        out_shape=[
            jax.ShapeDtypeStruct((B, _NI, EPG), jnp.float32),
            jax.ShapeDtypeStruct((B, _NI, NPGP), jnp.float32),
            jax.ShapeDtypeStruct((B, NPGP, D), jnp.float32),
        ],
    )(edge_attrs, W_edge, node_attrs, W_np, instr, foo,
      w_rel_score, w_node_score)

    ei3 = edge_indices.reshape(2, B, EPG)
    rs_sc = jnp.transpose(rs, (1, 0, 2))

    dist = _make_sc_loop(B, NPG, NPGP, EPG)(v, ei3, s, rs_sc, d0)

    out = pl.pallas_call(
        _final_body,
        out_shape=jax.ShapeDtypeStruct((B, OUT), jnp.float32),
    )(dist, aggv, enc, lin_W, lin_b)
    return out
